# Initial kernel scaffold; baseline (speedup 1.0000x reference)
import jax
import jax.numpy as jnp
from jax.experimental import pallas as pl


def kernel(x, edge_index, batch, W1, b1, W2, b2, Wl, bl):
    raise NotImplementedError("write your pallas kernel here")


# speedup vs baseline: 14.8591x; 14.8591x over previous
import functools

import jax
import jax.numpy as jnp
from jax import lax
from jax.experimental import pallas as pl
from jax.experimental.pallas import tpu as pltpu
from jax.experimental.pallas import tpu_sc as plsc

NC = 2
NS = 16
K = 80


def _sc_mesh():
    return plsc.VectorSubcoreMesh(core_axis_name="c", subcore_axis_name="s")


def _acc_rows(n):
    r = -(-n // NS)
    return -(-r // 40) * 40


def _make_deg_kernel(n, C):
    rows_per_sub = _acc_rows(n)
    acc_n = rows_per_sub * NS

    @functools.partial(
        pl.kernel,
        out_type=jax.ShapeDtypeStruct((NC, acc_n, 16), jnp.float32),
        mesh=_sc_mesh(),
        scratch_types=[
            pltpu.VMEM((C, K), jnp.int32),
            pltpu.VMEM((K, 16), jnp.float32),
            pltpu.VMEM((rows_per_sub, 16), jnp.float32),
            pltpu.VMEM_SHARED((acc_n, 16), jnp.float32),
        ],
        compiler_params=pltpu.CompilerParams(use_tc_tiling_on_sc=False),
    )
    def deg_k(dst_hbm, ones_hbm, zeros_hbm, out_hbm, dst_v, ones_v, stage_v, acc_sh):
        c = lax.axis_index("c")
        s = lax.axis_index("s")
        base = s * rows_per_sub
        pltpu.sync_copy(zeros_hbm, stage_v)
        pltpu.sync_copy(stage_v, acc_sh.at[pl.ds(base, rows_per_sub)])
        pltpu.sync_copy(ones_hbm, ones_v)
        pltpu.sync_copy(dst_hbm.at[c, s], dst_v)
        plsc.subcore_barrier()

        def body(j, carry):
            pltpu.sync_copy(ones_v, acc_sh.at[dst_v.at[j]], add=True)
            return carry

        lax.fori_loop(0, C, body, 0)
        plsc.subcore_barrier()
        pltpu.sync_copy(acc_sh.at[pl.ds(base, rows_per_sub)], stage_v)
        pltpu.sync_copy(stage_v, out_hbm.at[c, pl.ds(base, rows_per_sub)])

    return deg_k


def _make_agg_kernel(n, d, C2):
    dh = d // 2
    rows_per_sub = _acc_rows(n)
    acc_n = rows_per_sub * NS
    stage_rows = rows_per_sub // 5

    @functools.partial(
        pl.kernel,
        out_type=jax.ShapeDtypeStruct((NC, acc_n, dh), jnp.float32),
        mesh=_sc_mesh(),
        scratch_types=[
            pltpu.VMEM((C2, K), jnp.int32),
            pltpu.VMEM((C2, K), jnp.int32),
            pltpu.VMEM((2, K, dh), jnp.float32),
            pltpu.VMEM((stage_rows, dh), jnp.float32),
            pltpu.VMEM_SHARED((acc_n, dh), jnp.float32),
            pltpu.SemaphoreType.DMA,
        ],
        compiler_params=pltpu.CompilerParams(use_tc_tiling_on_sc=False),
    )
    def agg_k(table_hbm, src_hbm, dst_hbm, zeros_hbm, out_hbm,
              src_v, dst_v, rows_v, stage_v, acc_sh, sem):
        c = lax.axis_index("c")
        s = lax.axis_index("s")
        base = s * rows_per_sub
        pltpu.sync_copy(zeros_hbm, stage_v)
        for t in range(5):
            pltpu.sync_copy(stage_v, acc_sh.at[pl.ds(base + t * stage_rows, stage_rows)])
        pltpu.sync_copy(src_hbm.at[c, s], src_v)
        pltpu.sync_copy(dst_hbm.at[s], dst_v)
        plsc.subcore_barrier()

        def body(j, carry):
            pltpu.async_copy(table_hbm.at[src_v.at[j]], rows_v.at[0], sem).wait()
            pltpu.sync_copy(rows_v.at[0], acc_sh.at[dst_v.at[j]], add=True)
            return carry

        lax.fori_loop(0, C2, body, 0)
        plsc.subcore_barrier()
        for t in range(5):
            pltpu.sync_copy(acc_sh.at[pl.ds(base + t * stage_rows, stage_rows)], stage_v)
            pltpu.sync_copy(stage_v, out_hbm.at[c, pl.ds(base + t * stage_rows, stage_rows)])

    return agg_k


def _dinv_from(dacc0, dacc1):
    deg = dacc0[:, 0:1] + dacc1[:, 0:1] + 1.0
    return lax.rsqrt(deg)


def _tc_first(x, W1, dacc, bm):
    n, d_in = x.shape
    d_out = W1.shape[1]
    nblk = n // bm

    def body(x_ref, w_ref, dacc_ref, o_ref):
        dinv = _dinv_from(dacc_ref[0], dacc_ref[1])
        h = jnp.dot(x_ref[...], w_ref[...], preferred_element_type=jnp.float32)
        o_ref[...] = h * dinv

    return pl.pallas_call(
        body,
        grid=(nblk,),
        in_specs=[
            pl.BlockSpec((bm, d_in), lambda i: (i, 0)),
            pl.BlockSpec((d_in, d_out), lambda i: (0, 0)),
            pl.BlockSpec((NC, bm, 16), lambda i: (0, i, 0)),
        ],
        out_specs=pl.BlockSpec((bm, d_out), lambda i: (i, 0)),
        out_shape=jax.ShapeDtypeStruct((n, d_out), jnp.float32),
    )(x, W1, dacc)


def _tc_mid(sacc, hs, dacc, b, W, bm):
    n, d = hs.shape
    d_out = W.shape[1]
    nblk = n // bm

    def body(s_ref, hs_ref, dacc_ref, b_ref, w_ref, o_ref):
        dinv = _dinv_from(dacc_ref[0], dacc_ref[1])
        agg = jnp.concatenate([s_ref[0], s_ref[1]], axis=-1) + hs_ref[...]
        z = jnp.maximum(agg * dinv + b_ref[...], 0.0)
        o_ref[...] = jnp.dot(z, w_ref[...], preferred_element_type=jnp.float32) * dinv

    return pl.pallas_call(
        body,
        grid=(nblk,),
        in_specs=[
            pl.BlockSpec((NC, bm, d // 2), lambda i: (0, i, 0)),
            pl.BlockSpec((bm, d), lambda i: (i, 0)),
            pl.BlockSpec((NC, bm, 16), lambda i: (0, i, 0)),
            pl.BlockSpec((1, d), lambda i: (0, 0)),
            pl.BlockSpec((d, d_out), lambda i: (0, 0)),
        ],
        out_specs=pl.BlockSpec((bm, d_out), lambda i: (i, 0)),
        out_shape=jax.ShapeDtypeStruct((n, d_out), jnp.float32),
    )(sacc, hs, dacc, b, W)


def _tc_last(sacc, hs, dacc, b, Wl, bl, bm):
    n, d = hs.shape
    d_out = Wl.shape[1]
    nblk = n // bm
    inv_n = 1.0 / n

    def body(s_ref, hs_ref, dacc_ref, b_ref, w_ref, bl_ref, o_ref, pool_ref):
        i = pl.program_id(0)
        dinv = _dinv_from(dacc_ref[0], dacc_ref[1])
        agg = jnp.concatenate([s_ref[0], s_ref[1]], axis=-1) + hs_ref[...]
        z = jnp.maximum(agg * dinv + b_ref[...], 0.0)
        o_ref[...] = jnp.dot(z, w_ref[...], preferred_element_type=jnp.float32) + bl_ref[...]

        @pl.when(i == 0)
        def _():
            pool_ref[...] = jnp.zeros_like(pool_ref)

        pool_ref[...] += jnp.sum(z, axis=0, keepdims=True)

        @pl.when(i == nblk - 1)
        def _():
            pool_ref[...] = pool_ref[...] * inv_n

    return pl.pallas_call(
        body,
        grid=(nblk,),
        in_specs=[
            pl.BlockSpec((NC, bm, d // 2), lambda i: (0, i, 0)),
            pl.BlockSpec((bm, d), lambda i: (i, 0)),
            pl.BlockSpec((NC, bm, 16), lambda i: (0, i, 0)),
            pl.BlockSpec((1, d), lambda i: (0, 0)),
            pl.BlockSpec((d, d_out), lambda i: (0, 0)),
            pl.BlockSpec((1, d_out), lambda i: (0, 0)),
        ],
        out_specs=[
            pl.BlockSpec((bm, d_out), lambda i: (i, 0)),
            pl.BlockSpec((1, d_out), lambda i: (0, 0)),
        ],
        out_shape=[
            jax.ShapeDtypeStruct((n, d_out), jnp.float32),
            jax.ShapeDtypeStruct((1, d_out), jnp.float32),
        ],
    )(sacc, hs, dacc, b, Wl, bl)


def kernel(x, edge_index, batch, W1, b1, W2, b2, Wl, bl):
    n, d_in = x.shape
    e = edge_index.shape[1]
    d = W1.shape[1]
    C = e // (NC * NS * K)
    C2 = e // (NS * K)
    bm = 2000

    src = edge_index[0]
    dst = edge_index[1]
    dst_w = dst.reshape(NC, NS, C, K)
    src2 = jnp.stack([src * 2, src * 2 + 1]).reshape(NC, NS, C2, K)
    dst2 = dst.reshape(NS, C2, K)

    rows_per_sub = _acc_rows(n)
    zeros16 = jnp.zeros((rows_per_sub, 16), jnp.float32)
    ones_rows = jnp.zeros((K, 16), jnp.float32).at[:, 0].set(1.0)
    zeros_d = jnp.zeros((rows_per_sub // 5, d // 2), jnp.float32)

    deg_k = _make_deg_kernel(n, C)
    agg_k = _make_agg_kernel(n, d, C2)

    dacc = deg_k(dst_w, ones_rows, zeros16)

    b1r = b1.reshape(1, d)
    b2r = b2.reshape(1, d)
    blr = bl.reshape(1, Wl.shape[1])

    hs1 = _tc_first(x, W1, dacc, bm)
    s1 = agg_k(hs1.reshape(2 * n, d // 2), src2, dst2, zeros_d)
    hs2 = _tc_mid(s1, hs1, dacc, b1r, W2, bm)
    s2 = agg_k(hs2.reshape(2 * n, d // 2), src2, dst2, zeros_d)
    out, pooled = _tc_last(s2, hs2, dacc, b2r, Wl, blr, bm)
    return (out, pooled)


# speedup vs baseline: 21.8355x; 1.4695x over previous
import functools

import jax
import jax.numpy as jnp
from jax import lax
from jax.experimental import pallas as pl
from jax.experimental.pallas import tpu as pltpu
from jax.experimental.pallas import tpu_sc as plsc

NC = 2
NS = 16
KD = 80
KA = 128


def _sc_mesh():
    return plsc.VectorSubcoreMesh(core_axis_name="c", subcore_axis_name="s")


def _acc_rows(n):
    r = -(-n // NS)
    r = -(-r // 40) * 40
    if r * NS == n:
        r += 40
    return r


def _make_deg_kernel(n, C):
    rows_per_sub = _acc_rows(n)
    acc_n = rows_per_sub * NS

    @functools.partial(
        pl.kernel,
        out_type=jax.ShapeDtypeStruct((NC, acc_n, 16), jnp.float32),
        mesh=_sc_mesh(),
        scratch_types=[
            pltpu.VMEM((C, KD), jnp.int32),
            pltpu.VMEM((KD, 16), jnp.float32),
            pltpu.VMEM((rows_per_sub, 16), jnp.float32),
            pltpu.VMEM_SHARED((acc_n, 16), jnp.float32),
        ],
        compiler_params=pltpu.CompilerParams(use_tc_tiling_on_sc=False),
    )
    def deg_k(dst_hbm, ones_hbm, zeros_hbm, out_hbm, dst_v, ones_v, stage_v, acc_sh):
        c = lax.axis_index("c")
        s = lax.axis_index("s")
        base = s * rows_per_sub
        pltpu.sync_copy(zeros_hbm, stage_v)
        pltpu.sync_copy(stage_v, acc_sh.at[pl.ds(base, rows_per_sub)])
        pltpu.sync_copy(ones_hbm, ones_v)
        pltpu.sync_copy(dst_hbm.at[c, s], dst_v)
        plsc.subcore_barrier()

        def body(j, carry):
            pltpu.sync_copy(ones_v, acc_sh.at[dst_v.at[j]], add=True)
            return carry

        lax.fori_loop(0, C, body, 0)
        plsc.subcore_barrier()
        pltpu.sync_copy(acc_sh.at[pl.ds(base, rows_per_sub)], stage_v)
        pltpu.sync_copy(stage_v, out_hbm.at[c, pl.ds(base, rows_per_sub)])

    return deg_k


def _make_agg_kernel(n, d, C2):
    dh = d // 2
    rows_per_sub = _acc_rows(n)
    acc_n = rows_per_sub * NS
    stage_rows = rows_per_sub // 5

    @functools.partial(
        pl.kernel,
        out_type=jax.ShapeDtypeStruct((NC, acc_n, dh), jnp.float32),
        mesh=_sc_mesh(),
        scratch_types=[
            pltpu.VMEM((C2, KA), jnp.int32),
            pltpu.VMEM((C2, KA), jnp.int32),
            pltpu.VMEM((4, KA, dh), jnp.float32),
            pltpu.VMEM((stage_rows, dh), jnp.float32),
            pltpu.VMEM_SHARED((acc_n, dh), jnp.float32),
            pltpu.SemaphoreType.DMA((2,)),
            pltpu.SemaphoreType.DMA((2,)),
        ],
        compiler_params=pltpu.CompilerParams(use_tc_tiling_on_sc=False),
    )
    def agg_k(table_hbm, src_hbm, dst_hbm, zeros_hbm, out_hbm,
              src_v, dst_v, rows_v, stage_v, acc_sh, sg, ss):
        c = lax.axis_index("c")
        s = lax.axis_index("s")
        base = s * rows_per_sub
        pltpu.sync_copy(zeros_hbm, stage_v)
        for t in range(5):
            pltpu.sync_copy(stage_v, acc_sh.at[pl.ds(base + t * stage_rows, stage_rows)])
        pltpu.sync_copy(src_hbm.at[c, s], src_v)
        pltpu.sync_copy(dst_hbm.at[s], dst_v)
        plsc.subcore_barrier()

        def gath(j):
            pltpu.async_copy(table_hbm.at[src_v.at[j]], rows_v.at[j % 4],
                             sg.at[j % 2])

        def wait_gath(j):
            pltpu.make_async_copy(table_hbm.at[src_v.at[j]], rows_v.at[j % 4],
                                  sg.at[j % 2]).wait()

        def scat(j):
            pltpu.async_copy(rows_v.at[j % 4], acc_sh.at[dst_v.at[j]],
                             ss.at[j % 2], add=True)

        def wait_scat(j):
            pltpu.make_async_copy(rows_v.at[j % 4], acc_sh.at[dst_v.at[j]],
                                  ss.at[j % 2]).wait()

        gath(0)
        gath(1)

        def body(j, carry):
            wait_gath(j)

            @pl.when(j >= 2)
            def _():
                wait_scat(j - 2)

            scat(j)

            @pl.when(j + 2 < C2)
            def _():
                gath(j + 2)

            return carry

        lax.fori_loop(0, C2, body, 0)
        wait_scat(C2 - 2)
        wait_scat(C2 - 1)
        plsc.subcore_barrier()
        for t in range(5):
            pltpu.sync_copy(acc_sh.at[pl.ds(base + t * stage_rows, stage_rows)], stage_v)
            pltpu.sync_copy(stage_v, out_hbm.at[c, pl.ds(base + t * stage_rows, stage_rows)])

    return agg_k


def _dinv_from(dacc0, dacc1):
    deg = dacc0[:, 0:1] + dacc1[:, 0:1] + 1.0
    return lax.rsqrt(deg)


def _tc_first(x, W1, dacc, bm):
    n, d_in = x.shape
    d_out = W1.shape[1]
    nblk = n // bm

    def body(x_ref, w_ref, dacc_ref, o_ref):
        dinv = _dinv_from(dacc_ref[0], dacc_ref[1])
        h = jnp.dot(x_ref[...], w_ref[...], preferred_element_type=jnp.float32)
        o_ref[...] = h * dinv

    return pl.pallas_call(
        body,
        grid=(nblk,),
        in_specs=[
            pl.BlockSpec((bm, d_in), lambda i: (i, 0)),
            pl.BlockSpec((d_in, d_out), lambda i: (0, 0)),
            pl.BlockSpec((NC, bm, 16), lambda i: (0, i, 0)),
        ],
        out_specs=pl.BlockSpec((bm, d_out), lambda i: (i, 0)),
        out_shape=jax.ShapeDtypeStruct((n, d_out), jnp.float32),
    )(x, W1, dacc)


def _tc_mid(sacc, hs, dacc, b, W, bm):
    n, d = hs.shape
    d_out = W.shape[1]
    nblk = n // bm

    def body(s_ref, hs_ref, dacc_ref, b_ref, w_ref, o_ref):
        dinv = _dinv_from(dacc_ref[0], dacc_ref[1])
        agg = jnp.concatenate([s_ref[0], s_ref[1]], axis=-1) + hs_ref[...]
        z = jnp.maximum(agg * dinv + b_ref[...], 0.0)
        o_ref[...] = jnp.dot(z, w_ref[...], preferred_element_type=jnp.float32) * dinv

    return pl.pallas_call(
        body,
        grid=(nblk,),
        in_specs=[
            pl.BlockSpec((NC, bm, d // 2), lambda i: (0, i, 0)),
            pl.BlockSpec((bm, d), lambda i: (i, 0)),
            pl.BlockSpec((NC, bm, 16), lambda i: (0, i, 0)),
            pl.BlockSpec((1, d), lambda i: (0, 0)),
            pl.BlockSpec((d, d_out), lambda i: (0, 0)),
        ],
        out_specs=pl.BlockSpec((bm, d_out), lambda i: (i, 0)),
        out_shape=jax.ShapeDtypeStruct((n, d_out), jnp.float32),
    )(sacc, hs, dacc, b, W)


def _tc_last(sacc, hs, dacc, b, Wl, bl, bm):
    n, d = hs.shape
    d_out = Wl.shape[1]
    nblk = n // bm
    inv_n = 1.0 / n

    def body(s_ref, hs_ref, dacc_ref, b_ref, w_ref, bl_ref, o_ref, pool_ref):
        i = pl.program_id(0)
        dinv = _dinv_from(dacc_ref[0], dacc_ref[1])
        agg = jnp.concatenate([s_ref[0], s_ref[1]], axis=-1) + hs_ref[...]
        z = jnp.maximum(agg * dinv + b_ref[...], 0.0)
        o_ref[...] = jnp.dot(z, w_ref[...], preferred_element_type=jnp.float32) + bl_ref[...]

        @pl.when(i == 0)
        def _():
            pool_ref[...] = jnp.zeros_like(pool_ref)

        pool_ref[...] += jnp.sum(z, axis=0, keepdims=True)

        @pl.when(i == nblk - 1)
        def _():
            pool_ref[...] = pool_ref[...] * inv_n

    return pl.pallas_call(
        body,
        grid=(nblk,),
        in_specs=[
            pl.BlockSpec((NC, bm, d // 2), lambda i: (0, i, 0)),
            pl.BlockSpec((bm, d), lambda i: (i, 0)),
            pl.BlockSpec((NC, bm, 16), lambda i: (0, i, 0)),
            pl.BlockSpec((1, d), lambda i: (0, 0)),
            pl.BlockSpec((d, d_out), lambda i: (0, 0)),
            pl.BlockSpec((1, d_out), lambda i: (0, 0)),
        ],
        out_specs=[
            pl.BlockSpec((bm, d_out), lambda i: (i, 0)),
            pl.BlockSpec((1, d_out), lambda i: (0, 0)),
        ],
        out_shape=[
            jax.ShapeDtypeStruct((n, d_out), jnp.float32),
            jax.ShapeDtypeStruct((1, d_out), jnp.float32),
        ],
    )(sacc, hs, dacc, b, Wl, bl)


def kernel(x, edge_index, batch, W1, b1, W2, b2, Wl, bl):
    n, d_in = x.shape
    e = edge_index.shape[1]
    d = W1.shape[1]
    C = e // (NC * NS * KD)
    C2 = -(-e // (NS * KA))
    pad = NS * KA * C2 - e
    bm = 2000

    src = edge_index[0]
    dst = edge_index[1]
    dst_w = dst.reshape(NC, NS, C, KD)
    src2 = jnp.pad(jnp.stack([src * 2, src * 2 + 1]),
                   ((0, 0), (0, pad))).reshape(NC, NS, C2, KA)
    dst2 = jnp.pad(dst, (0, pad), constant_values=n).reshape(NS, C2, KA)

    rows_per_sub = _acc_rows(n)
    zeros16 = jnp.zeros((rows_per_sub, 16), jnp.float32)
    ones_rows = jnp.zeros((KD, 16), jnp.float32).at[:, 0].set(1.0)
    zeros_d = jnp.zeros((rows_per_sub // 5, d // 2), jnp.float32)

    deg_k = _make_deg_kernel(n, C)
    agg_k = _make_agg_kernel(n, d, C2)

    dacc = deg_k(dst_w, ones_rows, zeros16)

    b1r = b1.reshape(1, d)
    b2r = b2.reshape(1, d)
    blr = bl.reshape(1, Wl.shape[1])

    hs1 = _tc_first(x, W1, dacc, bm)
    s1 = agg_k(hs1.reshape(2 * n, d // 2), src2, dst2, zeros_d)
    hs2 = _tc_mid(s1, hs1, dacc, b1r, W2, bm)
    s2 = agg_k(hs2.reshape(2 * n, d // 2), src2, dst2, zeros_d)
    out, pooled = _tc_last(s2, hs2, dacc, b2r, Wl, blr, bm)
    return (out, pooled)


# speedup vs baseline: 22.9048x; 1.0490x over previous
import functools

import jax
import jax.numpy as jnp
from jax import lax
from jax.experimental import pallas as pl
from jax.experimental.pallas import tpu as pltpu
from jax.experimental.pallas import tpu_sc as plsc

NC = 2
NS = 16
KD = 80
KA = 128


def _sc_mesh():
    return plsc.VectorSubcoreMesh(core_axis_name="c", subcore_axis_name="s")


def _acc_rows(n):
    r = -(-n // NS)
    r = -(-r // 40) * 40
    if r * NS == n:
        r += 40
    return r


def _make_deg_kernel(n, C):
    rows_per_sub = _acc_rows(n)
    acc_n = rows_per_sub * NS

    @functools.partial(
        pl.kernel,
        out_type=jax.ShapeDtypeStruct((NC, acc_n, 16), jnp.float32),
        mesh=_sc_mesh(),
        scratch_types=[
            pltpu.VMEM((C, KD), jnp.int32),
            pltpu.VMEM((KD, 16), jnp.float32),
            pltpu.VMEM((rows_per_sub, 16), jnp.float32),
            pltpu.VMEM_SHARED((acc_n, 16), jnp.float32),
        ],
        compiler_params=pltpu.CompilerParams(use_tc_tiling_on_sc=False),
    )
    def deg_k(dst_hbm, ones_hbm, zeros_hbm, out_hbm, dst_v, ones_v, stage_v, acc_sh):
        c = lax.axis_index("c")
        s = lax.axis_index("s")
        base = s * rows_per_sub
        pltpu.sync_copy(zeros_hbm, stage_v)
        pltpu.sync_copy(stage_v, acc_sh.at[pl.ds(base, rows_per_sub)])
        pltpu.sync_copy(ones_hbm, ones_v)
        pltpu.sync_copy(dst_hbm.at[c, s], dst_v)
        plsc.subcore_barrier()

        def body(j, carry):
            pltpu.sync_copy(ones_v, acc_sh.at[dst_v.at[j]], add=True)
            return carry

        lax.fori_loop(0, C, body, 0)
        plsc.subcore_barrier()
        pltpu.sync_copy(acc_sh.at[pl.ds(base, rows_per_sub)], stage_v)
        pltpu.sync_copy(stage_v, out_hbm.at[c, pl.ds(base, rows_per_sub)])

    return deg_k


def _make_agg_kernel(n, d, C2):
    dh = d // 2
    rows_per_sub = _acc_rows(n)
    acc_n = rows_per_sub * NS
    stage_rows = rows_per_sub // 5

    @functools.partial(
        pl.kernel,
        out_type=jax.ShapeDtypeStruct((NC, acc_n, dh), jnp.float32),
        mesh=_sc_mesh(),
        scratch_types=[
            pltpu.VMEM((C2, KA), jnp.int32),
            pltpu.VMEM((C2, KA), jnp.int32),
            pltpu.VMEM((6, KA, dh), jnp.float32),
            pltpu.VMEM_SHARED((acc_n, dh), jnp.float32),
            pltpu.SemaphoreType.DMA((3,)),
            pltpu.SemaphoreType.DMA((3,)),
        ],
        compiler_params=pltpu.CompilerParams(use_tc_tiling_on_sc=False),
    )
    def agg_k(table_hbm, src_hbm, dst_hbm, zeros_hbm, out_hbm,
              src_v, dst_v, rows_v, acc_sh, sg, ss):
        c = lax.axis_index("c")
        s = lax.axis_index("s")
        base = s * rows_per_sub
        stage_v = rows_v.at[0]
        pltpu.sync_copy(zeros_hbm, stage_v)
        for t in range(5):
            pltpu.sync_copy(stage_v, acc_sh.at[pl.ds(base + t * stage_rows, stage_rows)])
        pltpu.sync_copy(src_hbm.at[c, s], src_v)
        pltpu.sync_copy(dst_hbm.at[s], dst_v)
        plsc.subcore_barrier()

        def gath(j):
            pltpu.async_copy(table_hbm.at[src_v.at[j]], rows_v.at[j % 6],
                             sg.at[j % 3])

        def wait_gath(j):
            pltpu.make_async_copy(table_hbm.at[src_v.at[j]], rows_v.at[j % 6],
                                  sg.at[j % 3]).wait()

        def scat(j):
            pltpu.async_copy(rows_v.at[j % 6], acc_sh.at[dst_v.at[j]],
                             ss.at[j % 3], add=True)

        def wait_scat(j):
            pltpu.make_async_copy(rows_v.at[j % 6], acc_sh.at[dst_v.at[j]],
                                  ss.at[j % 3]).wait()

        gath(0)
        gath(1)
        gath(2)

        def head(j, carry):
            wait_gath(j)
            scat(j)
            gath(j + 3)
            return carry

        def main(j, carry):
            wait_gath(j)
            wait_scat(j - 3)
            scat(j)
            gath(j + 3)
            return carry

        def tail(j, carry):
            wait_gath(j)
            wait_scat(j - 3)
            scat(j)
            return carry

        lax.fori_loop(0, 3, head, 0)
        lax.fori_loop(3, C2 - 3, main, 0)
        lax.fori_loop(C2 - 3, C2, tail, 0)
        wait_scat(C2 - 3)
        wait_scat(C2 - 2)
        wait_scat(C2 - 1)
        plsc.subcore_barrier()
        for t in range(5):
            pltpu.sync_copy(acc_sh.at[pl.ds(base + t * stage_rows, stage_rows)], stage_v)
            pltpu.sync_copy(stage_v, out_hbm.at[c, pl.ds(base + t * stage_rows, stage_rows)])

    return agg_k


def _dinv_from(dacc0, dacc1):
    deg = dacc0[:, 0:1] + dacc1[:, 0:1] + 1.0
    return lax.rsqrt(deg)


def _tc_first(x, W1, dacc, bm):
    n, d_in = x.shape
    d_out = W1.shape[1]
    nblk = n // bm

    def body(x_ref, w_ref, dacc_ref, o_ref):
        dinv = _dinv_from(dacc_ref[0], dacc_ref[1])
        h = jnp.dot(x_ref[...], w_ref[...], preferred_element_type=jnp.float32)
        o_ref[...] = h * dinv

    return pl.pallas_call(
        body,
        grid=(nblk,),
        in_specs=[
            pl.BlockSpec((bm, d_in), lambda i: (i, 0)),
            pl.BlockSpec((d_in, d_out), lambda i: (0, 0)),
            pl.BlockSpec((NC, bm, 16), lambda i: (0, i, 0)),
        ],
        out_specs=pl.BlockSpec((bm, d_out), lambda i: (i, 0)),
        out_shape=jax.ShapeDtypeStruct((n, d_out), jnp.float32),
    )(x, W1, dacc)


def _tc_mid(sacc, hs, dacc, b, W, bm):
    n, d = hs.shape
    d_out = W.shape[1]
    nblk = n // bm

    def body(s_ref, hs_ref, dacc_ref, b_ref, w_ref, o_ref):
        dinv = _dinv_from(dacc_ref[0], dacc_ref[1])
        agg = jnp.concatenate([s_ref[0], s_ref[1]], axis=-1) + hs_ref[...]
        z = jnp.maximum(agg * dinv + b_ref[...], 0.0)
        o_ref[...] = jnp.dot(z, w_ref[...], preferred_element_type=jnp.float32) * dinv

    return pl.pallas_call(
        body,
        grid=(nblk,),
        in_specs=[
            pl.BlockSpec((NC, bm, d // 2), lambda i: (0, i, 0)),
            pl.BlockSpec((bm, d), lambda i: (i, 0)),
            pl.BlockSpec((NC, bm, 16), lambda i: (0, i, 0)),
            pl.BlockSpec((1, d), lambda i: (0, 0)),
            pl.BlockSpec((d, d_out), lambda i: (0, 0)),
        ],
        out_specs=pl.BlockSpec((bm, d_out), lambda i: (i, 0)),
        out_shape=jax.ShapeDtypeStruct((n, d_out), jnp.float32),
    )(sacc, hs, dacc, b, W)


def _tc_last(sacc, hs, dacc, b, Wl, bl, bm):
    n, d = hs.shape
    d_out = Wl.shape[1]
    nblk = n // bm
    inv_n = 1.0 / n

    def body(s_ref, hs_ref, dacc_ref, b_ref, w_ref, bl_ref, o_ref, pool_ref):
        i = pl.program_id(0)
        dinv = _dinv_from(dacc_ref[0], dacc_ref[1])
        agg = jnp.concatenate([s_ref[0], s_ref[1]], axis=-1) + hs_ref[...]
        z = jnp.maximum(agg * dinv + b_ref[...], 0.0)
        o_ref[...] = jnp.dot(z, w_ref[...], preferred_element_type=jnp.float32) + bl_ref[...]

        @pl.when(i == 0)
        def _():
            pool_ref[...] = jnp.zeros_like(pool_ref)

        pool_ref[...] += jnp.sum(z, axis=0, keepdims=True)

        @pl.when(i == nblk - 1)
        def _():
            pool_ref[...] = pool_ref[...] * inv_n

    return pl.pallas_call(
        body,
        grid=(nblk,),
        in_specs=[
            pl.BlockSpec((NC, bm, d // 2), lambda i: (0, i, 0)),
            pl.BlockSpec((bm, d), lambda i: (i, 0)),
            pl.BlockSpec((NC, bm, 16), lambda i: (0, i, 0)),
            pl.BlockSpec((1, d), lambda i: (0, 0)),
            pl.BlockSpec((d, d_out), lambda i: (0, 0)),
            pl.BlockSpec((1, d_out), lambda i: (0, 0)),
        ],
        out_specs=[
            pl.BlockSpec((bm, d_out), lambda i: (i, 0)),
            pl.BlockSpec((1, d_out), lambda i: (0, 0)),
        ],
        out_shape=[
            jax.ShapeDtypeStruct((n, d_out), jnp.float32),
            jax.ShapeDtypeStruct((1, d_out), jnp.float32),
        ],
    )(sacc, hs, dacc, b, Wl, bl)


def kernel(x, edge_index, batch, W1, b1, W2, b2, Wl, bl):
    n, d_in = x.shape
    e = edge_index.shape[1]
    d = W1.shape[1]
    C = e // (NC * NS * KD)
    C2 = -(-e // (NS * KA))
    pad = NS * KA * C2 - e
    bm = 2000

    src = edge_index[0]
    dst = edge_index[1]
    dst_w = dst.reshape(NC, NS, C, KD)
    src2 = jnp.pad(jnp.stack([src * 2, src * 2 + 1]),
                   ((0, 0), (0, pad))).reshape(NC, NS, C2, KA)
    dst2 = jnp.pad(dst, (0, pad), constant_values=n).reshape(NS, C2, KA)

    rows_per_sub = _acc_rows(n)
    zeros16 = jnp.zeros((rows_per_sub, 16), jnp.float32)
    ones_rows = jnp.zeros((KD, 16), jnp.float32).at[:, 0].set(1.0)
    zeros_d = jnp.zeros((rows_per_sub // 5, d // 2), jnp.float32)

    deg_k = _make_deg_kernel(n, C)
    agg_k = _make_agg_kernel(n, d, C2)

    dacc = deg_k(dst_w, ones_rows, zeros16)

    b1r = b1.reshape(1, d)
    b2r = b2.reshape(1, d)
    blr = bl.reshape(1, Wl.shape[1])

    hs1 = _tc_first(x, W1, dacc, bm)
    s1 = agg_k(hs1.reshape(2 * n, d // 2), src2, dst2, zeros_d)
    hs2 = _tc_mid(s1, hs1, dacc, b1r, W2, bm)
    s2 = agg_k(hs2.reshape(2 * n, d // 2), src2, dst2, zeros_d)
    out, pooled = _tc_last(s2, hs2, dacc, b2r, Wl, blr, bm)
    return (out, pooled)
